# baseline (speedup 1.0000x reference)
import functools

import jax
import jax.numpy as jnp
from jax import lax
from jax.experimental import pallas as pl
from jax.experimental.pallas import tpu as pltpu
from jax.experimental.pallas import tpu_sc as plsc

_NUM_NODES = 1000000
_NUM_RELATIONS = 1000
_HIDDEN = 64
_BATCH = 16384

_NC = 2
_NS = 16
_NW = _NC * _NS
_ROWS_PER_W = _BATCH // _NW
_CHUNK = 128
_NCHUNK = _ROWS_PER_W // _CHUNK


def _nrsqrt(x):
    i = lax.bitcast_convert_type(x, jnp.int32)
    y = lax.bitcast_convert_type(jnp.int32(0x5F3759DF) - (i >> 1), jnp.float32)
    for _ in range(3):
        y = y * (1.5 - 0.5 * x * y * y)
    return y


def _body(src_r, rel_r, dst_r, ent_r, relt_r, out_r,
          s_idx, r_idx, d_idx, s_rows, r_rows, d_rows, scores, sem):
    cid = lax.axis_index("c")
    sid = lax.axis_index("s")
    wid = sid * _NC + cid
    base4 = wid * _NCHUNK

    pltpu.sync_copy(src_r.at[pl.ds(base4, _NCHUNK)], s_idx)
    pltpu.sync_copy(rel_r.at[pl.ds(base4, _NCHUNK)], r_idx)
    pltpu.sync_copy(dst_r.at[pl.ds(base4, _NCHUNK)], d_idx)

    copies = []
    for j in range(_NCHUNK):
        copies.append(pltpu.async_copy(ent_r.at[s_idx.at[j]], s_rows.at[j], sem))
        copies.append(pltpu.async_copy(ent_r.at[d_idx.at[j]], d_rows.at[j], sem))
        copies.append(pltpu.async_copy(relt_r.at[r_idx.at[j]], r_rows.at[j], sem))
    for cp in copies:
        cp.wait()

    lanes = lax.iota(jnp.int32, 16)

    def grp_body(g, carry):
        j = g // (_CHUNK // 16)
        rb = (g % (_CHUNK // 16)) * 16
        jv = jnp.full((16,), 0, jnp.int32) + j
        rv = rb + lanes

        def h_body(h, acc):
            ss, dd, rr, sr, sd, rd = acc
            hv = (lanes + h) & (_HIDDEN - 1)
            sv = plsc.load_gather(s_rows, [jv, rv, hv])
            dv = plsc.load_gather(d_rows, [jv, rv, hv])
            rlv = plsc.load_gather(r_rows, [jv, rv, hv])
            return (ss + sv * sv, dd + dv * dv, rr + rlv * rlv,
                    sr + sv * rlv, sd + sv * dv, rd + dv * rlv)

        z = jnp.zeros((16,), jnp.float32)
        ss, dd, rr, sr, sd, rd = lax.fori_loop(
            0, _HIDDEN, h_body, (z, z, z, z, z, z))

        sl = pl.ds(g * 16, 16)
        ns = ss * _nrsqrt(ss)
        nd = dd * _nrsqrt(dd)
        a = 1.0 / jnp.maximum(ns, 1e-12)
        b = 1.0 / jnp.maximum(nd, 1e-12)
        sq = rr + ss * (a * a) + dd * (b * b) \
            + 2.0 * (sr * a - rd * b - sd * (a * b))
        sq = jnp.maximum(sq, 0.0)
        scores[sl] = sq * _nrsqrt(sq)
        return carry

    lax.fori_loop(0, _ROWS_PER_W // 16, grp_body, 0)

    pltpu.sync_copy(scores, out_r.at[pl.ds(wid * _ROWS_PER_W, _ROWS_PER_W)])


_mesh = plsc.VectorSubcoreMesh(core_axis_name="c", subcore_axis_name="s")

_sc_call = functools.partial(
    pl.kernel,
    mesh=_mesh,
    compiler_params=pltpu.CompilerParams(
        use_tc_tiling_on_sc=False, needs_layout_passes=False),
    out_type=jax.ShapeDtypeStruct((_BATCH,), jnp.float32),
    scratch_types=[
        pltpu.VMEM((_NCHUNK, _CHUNK), jnp.int32),
        pltpu.VMEM((_NCHUNK, _CHUNK), jnp.int32),
        pltpu.VMEM((_NCHUNK, _CHUNK), jnp.int32),
        pltpu.VMEM((_NCHUNK, _CHUNK, _HIDDEN), jnp.float32),
        pltpu.VMEM((_NCHUNK, _CHUNK, _HIDDEN), jnp.float32),
        pltpu.VMEM((_NCHUNK, _CHUNK, _HIDDEN), jnp.float32),
        pltpu.VMEM((_ROWS_PER_W,), jnp.float32),
        pltpu.SemaphoreType.DMA,
    ],
)(_body)


@jax.jit
def kernel(src, rel, dst, entity_embedding, relation_embedding):
    src2 = src.astype(jnp.int32).reshape(_BATCH // _CHUNK, _CHUNK)
    rel2 = rel.astype(jnp.int32).reshape(_BATCH // _CHUNK, _CHUNK)
    dst2 = dst.astype(jnp.int32).reshape(_BATCH // _CHUNK, _CHUNK)
    return _sc_call(src2, rel2, dst2, entity_embedding, relation_embedding)
